# baseline (speedup 1.0000x reference)
import functools

import jax
import jax.numpy as jnp
from jax.experimental import pallas as pl
from jax.experimental.pallas import tpu as pltpu

TILE_N = 2048


def _embed_kernel(day_ref, week_ref, table_ref, out_ref, *, time_steps, n_weeks):
    tile_n = out_ref.shape[-1]
    nw_pad = table_ref.shape[-1] - time_steps

    day = day_ref[...]
    week = week_ref[...]

    day_idx = jnp.clip((day * float(time_steps)).astype(jnp.int32),
                       0, time_steps - 1)
    week_idx = jnp.clip(week.astype(jnp.int32), 0, n_weeks - 1)

    iota_d = jax.lax.broadcasted_iota(jnp.int32, (time_steps, tile_n), 0)
    iota_w = jax.lax.broadcasted_iota(jnp.int32, (nw_pad, tile_n), 0)
    onehot = jnp.concatenate(
        [(iota_d == day_idx).astype(jnp.float32),
         (iota_w == week_idx).astype(jnp.float32)], axis=0)

    out_ref[...] = jnp.dot(table_ref[...], onehot,
                           preferred_element_type=jnp.float32)


def kernel(x, time_day, time_week):
    B, T, N, C = x.shape
    time_steps, F = time_day.shape
    n_weeks = time_week.shape[0]

    nw_pad = ((n_weeks + 7) // 8) * 8
    table_t = jnp.zeros((F, time_steps + nw_pad), jnp.float32)
    table_t = table_t.at[:, :time_steps].set(time_day.astype(jnp.float32).T)
    table_t = table_t.at[:, time_steps:time_steps + n_weeks].set(
        time_week.astype(jnp.float32).T)

    body = functools.partial(_embed_kernel,
                             time_steps=time_steps, n_weeks=n_weeks)

    day = x[:, -1:, :, 1].astype(jnp.float32)
    week = x[:, -1:, :, 2].astype(jnp.float32)
    n_pad = ((N + TILE_N - 1) // TILE_N) * TILE_N
    if n_pad != N:
        day = jnp.pad(day, ((0, 0), (0, 0), (0, n_pad - N)))
        week = jnp.pad(week, ((0, 0), (0, 0), (0, n_pad - N)))

    n_tiles = n_pad // TILE_N
    out = pl.pallas_call(
        body,
        out_shape=jax.ShapeDtypeStruct((B, F, n_pad), jnp.float32),
        grid=(B, n_tiles),
        in_specs=[
            pl.BlockSpec((None, 1, TILE_N), lambda b, n: (b, 0, n)),
            pl.BlockSpec((None, 1, TILE_N), lambda b, n: (b, 0, n)),
            pl.BlockSpec((F, time_steps + nw_pad), lambda b, n: (0, 0)),
        ],
        out_specs=pl.BlockSpec((None, F, TILE_N), lambda b, n: (b, 0, n)),
        compiler_params=pltpu.CompilerParams(
            dimension_semantics=("parallel", "parallel")),
    )(day, week, table_t)

    return jnp.transpose(out[:, :, :N][:, None, :, :], (0, 2, 3, 1))


# speedup vs baseline: 1.0013x; 1.0013x over previous
import functools

import jax
import jax.numpy as jnp
from jax.experimental import pallas as pl
from jax.experimental.pallas import tpu as pltpu

TILE_N = 2048


def _embed_kernel(day_ref, week_ref, table_ref, out_ref, *, time_steps, n_weeks):
    tile_n = out_ref.shape[-1]
    nw_pad = table_ref.shape[-1] - time_steps

    day = day_ref[...]
    week = week_ref[...]

    day_idx = jnp.clip((day * float(time_steps)).astype(jnp.int32),
                       0, time_steps - 1)
    week_idx = jnp.clip(week.astype(jnp.int32), 0, n_weeks - 1)

    iota_d = jax.lax.broadcasted_iota(jnp.int32, (time_steps, tile_n), 0)
    iota_w = jax.lax.broadcasted_iota(jnp.int32, (nw_pad, tile_n), 0)
    onehot = jnp.concatenate(
        [(iota_d == day_idx).astype(jnp.float32),
         (iota_w == week_idx).astype(jnp.float32)], axis=0)

    out_ref[...] = jnp.dot(table_ref[...], onehot,
                           preferred_element_type=jnp.float32)


def kernel(x, time_day, time_week):
    B, T, N, C = x.shape
    time_steps, F = time_day.shape
    n_weeks = time_week.shape[0]

    nw_pad = ((n_weeks + 7) // 8) * 8
    table_t = jnp.zeros((F, time_steps + nw_pad), jnp.float32)
    table_t = table_t.at[:, :time_steps].set(time_day.astype(jnp.float32).T)
    table_t = table_t.at[:, time_steps:time_steps + n_weeks].set(
        time_week.astype(jnp.float32).T)

    body = functools.partial(_embed_kernel,
                             time_steps=time_steps, n_weeks=n_weeks)

    day = x[:, -1:, :, 1].astype(jnp.float32)
    week = x[:, -1:, :, 2].astype(jnp.float32)
    n_pad = ((N + TILE_N - 1) // TILE_N) * TILE_N
    if n_pad != N:
        day = jnp.pad(day, ((0, 0), (0, 0), (0, n_pad - N)))
        week = jnp.pad(week, ((0, 0), (0, 0), (0, n_pad - N)))

    n_tiles = n_pad // TILE_N
    out = pl.pallas_call(
        body,
        out_shape=jax.ShapeDtypeStruct((B, 1, F, n_pad), jnp.float32),
        grid=(B, n_tiles),
        in_specs=[
            pl.BlockSpec((None, 1, TILE_N), lambda b, n: (b, 0, n)),
            pl.BlockSpec((None, 1, TILE_N), lambda b, n: (b, 0, n)),
            pl.BlockSpec((F, time_steps + nw_pad), lambda b, n: (0, 0)),
        ],
        out_specs=pl.BlockSpec((None, None, F, TILE_N), lambda b, n: (b, 0, 0, n)),
        compiler_params=pltpu.CompilerParams(
            dimension_semantics=("parallel", "parallel")),
    )(day, week, table_t)

    return jnp.transpose(out[:, :, :, :N], (0, 2, 3, 1))


# speedup vs baseline: 1.9417x; 1.9391x over previous
import functools

import jax
import jax.numpy as jnp
from jax.experimental import pallas as pl
from jax.experimental.pallas import tpu as pltpu

TILE_N = 2048


def _embed_kernel(day_ref, week_ref, table_ref, out_ref, *, time_steps, n_weeks):
    tile_n = out_ref.shape[-1]
    nw_pad = table_ref.shape[-1] - time_steps

    day = day_ref[...]
    week = week_ref[...]

    day_idx = jnp.clip((day * float(time_steps)).astype(jnp.int32),
                       0, time_steps - 1)
    week_idx = jnp.clip(week.astype(jnp.int32), 0, n_weeks - 1)

    iota_d = jax.lax.broadcasted_iota(jnp.int32, (time_steps, tile_n), 0)
    iota_w = jax.lax.broadcasted_iota(jnp.int32, (nw_pad, tile_n), 0)
    onehot = jnp.concatenate(
        [(iota_d == day_idx).astype(jnp.float32),
         (iota_w == week_idx).astype(jnp.float32)], axis=0)

    out_ref[...] = jnp.dot(table_ref[...], onehot,
                           preferred_element_type=jnp.float32)[:, None, :]


def kernel(x, time_day, time_week):
    B, T, N, C = x.shape
    time_steps, F = time_day.shape
    n_weeks = time_week.shape[0]

    nw_pad = ((n_weeks + 7) // 8) * 8
    table_t = jnp.zeros((F, time_steps + nw_pad), jnp.float32)
    table_t = table_t.at[:, :time_steps].set(time_day.astype(jnp.float32).T)
    table_t = table_t.at[:, time_steps:time_steps + n_weeks].set(
        time_week.astype(jnp.float32).T)

    body = functools.partial(_embed_kernel,
                             time_steps=time_steps, n_weeks=n_weeks)

    day = x[:, -1:, :, 1].astype(jnp.float32)
    week = x[:, -1:, :, 2].astype(jnp.float32)
    n_pad = ((N + TILE_N - 1) // TILE_N) * TILE_N
    if n_pad != N:
        day = jnp.pad(day, ((0, 0), (0, 0), (0, n_pad - N)))
        week = jnp.pad(week, ((0, 0), (0, 0), (0, n_pad - N)))

    n_tiles = n_pad // TILE_N
    out = pl.pallas_call(
        body,
        out_shape=jax.ShapeDtypeStruct((B, F, 1, n_pad), jnp.float32),
        grid=(B, n_tiles),
        in_specs=[
            pl.BlockSpec((None, 1, TILE_N), lambda b, n: (b, 0, n)),
            pl.BlockSpec((None, 1, TILE_N), lambda b, n: (b, 0, n)),
            pl.BlockSpec((F, time_steps + nw_pad), lambda b, n: (0, 0)),
        ],
        out_specs=pl.BlockSpec((None, F, 1, TILE_N), lambda b, n: (b, 0, 0, n)),
        compiler_params=pltpu.CompilerParams(
            dimension_semantics=("parallel", "parallel")),
    )(day, week, table_t)

    return jnp.transpose(out[:, :, :, :N], (0, 1, 3, 2))


# speedup vs baseline: 2.5582x; 1.3175x over previous
import functools

import jax
import jax.numpy as jnp
from jax.experimental import pallas as pl
from jax.experimental.pallas import tpu as pltpu

TILE_N = 2048


def _embed_kernel(day_ref, week_ref, table_ref, out_ref, *, time_steps, n_weeks):
    bb = out_ref.shape[0]
    tile_n = out_ref.shape[-1]
    nw_pad = table_ref.shape[-1] - time_steps

    iota_d = jax.lax.broadcasted_iota(jnp.int32, (time_steps, tile_n), 0)
    iota_w = jax.lax.broadcasted_iota(jnp.int32, (nw_pad, tile_n), 0)

    for i in range(bb):
        day = day_ref[i]
        week = week_ref[i]

        day_idx = jnp.clip((day * float(time_steps)).astype(jnp.int32),
                           0, time_steps - 1)
        week_idx = jnp.clip(week.astype(jnp.int32), 0, n_weeks - 1)

        onehot = jnp.concatenate(
            [(iota_d == day_idx).astype(jnp.float32),
             (iota_w == week_idx).astype(jnp.float32)], axis=0)

        out_ref[i] = jnp.dot(table_ref[...], onehot,
                             preferred_element_type=jnp.float32)[:, None, :]


def kernel(x, time_day, time_week):
    B, T, N, C = x.shape
    time_steps, F = time_day.shape
    n_weeks = time_week.shape[0]

    nw_pad = ((n_weeks + 7) // 8) * 8
    table_t = jnp.zeros((F, time_steps + nw_pad), jnp.float32)
    table_t = table_t.at[:, :time_steps].set(time_day.astype(jnp.float32).T)
    table_t = table_t.at[:, time_steps:time_steps + n_weeks].set(
        time_week.astype(jnp.float32).T)

    body = functools.partial(_embed_kernel,
                             time_steps=time_steps, n_weeks=n_weeks)

    day = x[:, -1:, :, 1].astype(jnp.float32)
    week = x[:, -1:, :, 2].astype(jnp.float32)
    n_pad = ((N + TILE_N - 1) // TILE_N) * TILE_N
    if n_pad != N:
        day = jnp.pad(day, ((0, 0), (0, 0), (0, n_pad - N)))
        week = jnp.pad(week, ((0, 0), (0, 0), (0, n_pad - N)))

    n_tiles = n_pad // TILE_N
    bb = 2 if (B % 2 == 0 and n_tiles == 1) else 1
    out = pl.pallas_call(
        body,
        out_shape=jax.ShapeDtypeStruct((B, F, 1, n_pad), jnp.float32),
        grid=(B // bb, n_tiles),
        in_specs=[
            pl.BlockSpec((bb, 1, TILE_N), lambda b, n: (b, 0, n)),
            pl.BlockSpec((bb, 1, TILE_N), lambda b, n: (b, 0, n)),
            pl.BlockSpec((F, time_steps + nw_pad), lambda b, n: (0, 0)),
        ],
        out_specs=pl.BlockSpec((bb, F, 1, TILE_N), lambda b, n: (b, 0, 0, n)),
        compiler_params=pltpu.CompilerParams(
            dimension_semantics=("parallel", "parallel")),
    )(day, week, table_t)

    return jnp.transpose(out[:, :, :, :N], (0, 1, 3, 2))


# speedup vs baseline: 2.9666x; 1.1596x over previous
import functools

import jax
import jax.numpy as jnp
from jax.experimental import pallas as pl
from jax.experimental.pallas import tpu as pltpu

TILE_N = 2048


def _embed_kernel(day_ref, week_ref, table_ref, out_ref, *, time_steps, n_weeks):
    bb = out_ref.shape[0]
    tile_n = out_ref.shape[-1]
    nw_pad = table_ref.shape[-1] - time_steps

    iota_d = jax.lax.broadcasted_iota(jnp.int32, (time_steps, tile_n), 0)
    iota_w = jax.lax.broadcasted_iota(jnp.int32, (nw_pad, tile_n), 0)

    for i in range(bb):
        day = day_ref[i]
        week = week_ref[i]

        day_idx = jnp.clip((day * float(time_steps)).astype(jnp.int32),
                           0, time_steps - 1)
        week_idx = jnp.clip(week.astype(jnp.int32), 0, n_weeks - 1)

        onehot = jnp.concatenate(
            [(iota_d == day_idx).astype(jnp.float32),
             (iota_w == week_idx).astype(jnp.float32)], axis=0)

        out_ref[i] = jnp.dot(table_ref[...], onehot,
                             preferred_element_type=jnp.float32)[:, None, :]


def kernel(x, time_day, time_week):
    B, T, N, C = x.shape
    time_steps, F = time_day.shape
    n_weeks = time_week.shape[0]

    nw_pad = ((n_weeks + 7) // 8) * 8
    table_t = jnp.zeros((F, time_steps + nw_pad), jnp.float32)
    table_t = table_t.at[:, :time_steps].set(time_day.astype(jnp.float32).T)
    table_t = table_t.at[:, time_steps:time_steps + n_weeks].set(
        time_week.astype(jnp.float32).T)

    body = functools.partial(_embed_kernel,
                             time_steps=time_steps, n_weeks=n_weeks)

    day = x[:, -1:, :, 1].astype(jnp.float32)
    week = x[:, -1:, :, 2].astype(jnp.float32)
    n_pad = ((N + TILE_N - 1) // TILE_N) * TILE_N
    if n_pad != N:
        day = jnp.pad(day, ((0, 0), (0, 0), (0, n_pad - N)))
        week = jnp.pad(week, ((0, 0), (0, 0), (0, n_pad - N)))

    n_tiles = n_pad // TILE_N
    bb = 4 if (B % 4 == 0 and n_tiles == 1) else 1
    out = pl.pallas_call(
        body,
        out_shape=jax.ShapeDtypeStruct((B, F, 1, n_pad), jnp.float32),
        grid=(B // bb, n_tiles),
        in_specs=[
            pl.BlockSpec((bb, 1, TILE_N), lambda b, n: (b, 0, n)),
            pl.BlockSpec((bb, 1, TILE_N), lambda b, n: (b, 0, n)),
            pl.BlockSpec((F, time_steps + nw_pad), lambda b, n: (0, 0)),
        ],
        out_specs=pl.BlockSpec((bb, F, 1, TILE_N), lambda b, n: (b, 0, 0, n)),
        compiler_params=pltpu.CompilerParams(
            dimension_semantics=("parallel", "parallel")),
    )(day, week, table_t)

    return jnp.transpose(out[:, :, :, :N], (0, 1, 3, 2))


# speedup vs baseline: 3.0340x; 1.0227x over previous
import functools

import jax
import jax.numpy as jnp
from jax.experimental import pallas as pl
from jax.experimental.pallas import tpu as pltpu

TILE_N = 2048


def _embed_kernel(day_ref, week_ref, table_ref, out_ref, *, time_steps, n_weeks):
    bb = out_ref.shape[0]
    tile_n = out_ref.shape[-1]
    nw_pad = table_ref.shape[-1] - time_steps

    iota_d = jax.lax.broadcasted_iota(jnp.int32, (time_steps, tile_n), 0)
    iota_w = jax.lax.broadcasted_iota(jnp.int32, (nw_pad, tile_n), 0)

    for i in range(bb):
        day = day_ref[i]
        week = week_ref[i]

        day_idx = jnp.clip((day * float(time_steps)).astype(jnp.int32),
                           0, time_steps - 1)
        week_idx = jnp.clip(week.astype(jnp.int32), 0, n_weeks - 1)

        onehot = jnp.concatenate(
            [(iota_d == day_idx).astype(jnp.float32),
             (iota_w == week_idx).astype(jnp.float32)], axis=0)

        out_ref[i] = jnp.dot(table_ref[...], onehot,
                             preferred_element_type=jnp.float32)[:, None, :]


def kernel(x, time_day, time_week):
    B, T, N, C = x.shape
    time_steps, F = time_day.shape
    n_weeks = time_week.shape[0]

    nw_pad = ((n_weeks + 7) // 8) * 8
    table_t = jnp.zeros((F, time_steps + nw_pad), jnp.float32)
    table_t = table_t.at[:, :time_steps].set(time_day.astype(jnp.float32).T)
    table_t = table_t.at[:, time_steps:time_steps + n_weeks].set(
        time_week.astype(jnp.float32).T)

    body = functools.partial(_embed_kernel,
                             time_steps=time_steps, n_weeks=n_weeks)

    day = x[:, -1:, :, 1].astype(jnp.float32)
    week = x[:, -1:, :, 2].astype(jnp.float32)
    n_pad = ((N + TILE_N - 1) // TILE_N) * TILE_N
    if n_pad != N:
        day = jnp.pad(day, ((0, 0), (0, 0), (0, n_pad - N)))
        week = jnp.pad(week, ((0, 0), (0, 0), (0, n_pad - N)))

    n_tiles = n_pad // TILE_N
    bb = 8 if (B % 8 == 0 and n_tiles == 1) else 1
    out = pl.pallas_call(
        body,
        out_shape=jax.ShapeDtypeStruct((B, F, 1, n_pad), jnp.float32),
        grid=(B // bb, n_tiles),
        in_specs=[
            pl.BlockSpec((bb, 1, TILE_N), lambda b, n: (b, 0, n)),
            pl.BlockSpec((bb, 1, TILE_N), lambda b, n: (b, 0, n)),
            pl.BlockSpec((F, time_steps + nw_pad), lambda b, n: (0, 0)),
        ],
        out_specs=pl.BlockSpec((bb, F, 1, TILE_N), lambda b, n: (b, 0, 0, n)),
        compiler_params=pltpu.CompilerParams(
            dimension_semantics=("parallel", "parallel")),
    )(day, week, table_t)

    return jnp.transpose(out[:, :, :, :N], (0, 1, 3, 2))


# speedup vs baseline: 3.2877x; 1.0836x over previous
import functools

import jax
import jax.numpy as jnp
from jax.experimental import pallas as pl
from jax.experimental.pallas import tpu as pltpu

TILE_N = 2048


def _embed_kernel(day_ref, week_ref, table_ref, out_ref, *, time_steps, n_weeks):
    bb = out_ref.shape[0]
    tile_n = out_ref.shape[-1]
    nw_pad = table_ref.shape[0] - time_steps

    iota_d = jax.lax.broadcasted_iota(jnp.int32, (time_steps, tile_n), 0)
    iota_w = jax.lax.broadcasted_iota(jnp.int32, (nw_pad, tile_n), 0)

    for i in range(bb):
        day = day_ref[i]
        week = week_ref[i]

        day_idx = jnp.clip((day * float(time_steps)).astype(jnp.int32),
                           0, time_steps - 1)
        week_idx = jnp.clip(week.astype(jnp.int32), 0, n_weeks - 1)

        onehot = jnp.concatenate(
            [(iota_d == day_idx).astype(jnp.bfloat16),
             (iota_w == week_idx).astype(jnp.bfloat16)], axis=0)

        res = jax.lax.dot_general(
            table_ref[...], onehot, (((0,), (0,)), ((), ())),
            preferred_element_type=jnp.float32)
        out_ref[i] = res[:, None, :]


def kernel(x, time_day, time_week):
    B, T, N, C = x.shape
    time_steps, F = time_day.shape
    n_weeks = time_week.shape[0]

    nw_pad = ((n_weeks + 7) // 8) * 8
    table = jnp.concatenate(
        [time_day, time_week,
         jnp.zeros((nw_pad - n_weeks, F), time_week.dtype)],
        axis=0).astype(jnp.bfloat16)

    body = functools.partial(_embed_kernel,
                             time_steps=time_steps, n_weeks=n_weeks)

    day = x[:, -1:, :, 1].astype(jnp.float32)
    week = x[:, -1:, :, 2].astype(jnp.float32)
    n_pad = ((N + TILE_N - 1) // TILE_N) * TILE_N
    if n_pad != N:
        day = jnp.pad(day, ((0, 0), (0, 0), (0, n_pad - N)))
        week = jnp.pad(week, ((0, 0), (0, 0), (0, n_pad - N)))

    n_tiles = n_pad // TILE_N
    bb = 8 if (B % 8 == 0 and n_tiles == 1) else 1
    out = pl.pallas_call(
        body,
        out_shape=jax.ShapeDtypeStruct((B, F, 1, n_pad), jnp.float32),
        grid=(B // bb, n_tiles),
        in_specs=[
            pl.BlockSpec((bb, 1, TILE_N), lambda b, n: (b, 0, n)),
            pl.BlockSpec((bb, 1, TILE_N), lambda b, n: (b, 0, n)),
            pl.BlockSpec((time_steps + nw_pad, F), lambda b, n: (0, 0)),
        ],
        out_specs=pl.BlockSpec((bb, F, 1, TILE_N), lambda b, n: (b, 0, 0, n)),
        compiler_params=pltpu.CompilerParams(
            dimension_semantics=("parallel", "parallel")),
    )(day, week, table)

    return jnp.transpose(out[:, :, :, :N], (0, 1, 3, 2))
